# baseline (speedup 1.0000x reference)
import functools

import jax
import jax.numpy as jnp
from jax.experimental import pallas as pl
from jax.experimental.pallas import tpu as pltpu

N = 4096
D_IN = 64
D_OUT = 64
BM = 256


def _gcn_kernel(infeatn_ref, adj_ref, w_ref, b_ref, out_ref, support_ref):
    @pl.when(pl.program_id(0) == 0)
    def _():
        support_ref[...] = jnp.dot(
            infeatn_ref[...], w_ref[...], preferred_element_type=jnp.float32
        )

    out_ref[...] = (
        jnp.dot(adj_ref[...], support_ref[...], preferred_element_type=jnp.float32)
        + b_ref[...]
    )


@jax.jit
def kernel(infeatn, adj, W, b):
    b2 = b.reshape(1, D_OUT)
    grid = (N // BM,)
    return pl.pallas_call(
        _gcn_kernel,
        grid=grid,
        in_specs=[
            pl.BlockSpec((N, D_IN), lambda i: (0, 0)),
            pl.BlockSpec((BM, N), lambda i: (i, 0)),
            pl.BlockSpec((D_IN, D_OUT), lambda i: (0, 0)),
            pl.BlockSpec((1, D_OUT), lambda i: (0, 0)),
        ],
        out_specs=pl.BlockSpec((BM, D_OUT), lambda i: (i, 0)),
        out_shape=jax.ShapeDtypeStruct((N, D_OUT), jnp.float32),
        scratch_shapes=[pltpu.VMEM((N, D_OUT), jnp.float32)],
    )(infeatn, adj, W, b2)


# speedup vs baseline: 1.0857x; 1.0857x over previous
import jax
import jax.numpy as jnp
from jax.experimental import pallas as pl
from jax.experimental.pallas import tpu as pltpu

N = 4096
D_IN = 64
D_OUT = 64
BM = 256
NBUF = 4
NSTEPS = N // BM


def _gcn_kernel(infeatn_ref, adj_hbm, w_ref, b_ref, out_ref, ring, support_ref, sems):
    support_ref[...] = jnp.dot(
        infeatn_ref[...], w_ref[...], preferred_element_type=jnp.float32
    )

    def start_copy(step):
        slot = jax.lax.rem(step, NBUF)
        pltpu.make_async_copy(
            adj_hbm.at[pl.ds(step * BM, BM), :],
            ring.at[slot],
            sems.at[slot],
        ).start()

    for k in range(NBUF):
        start_copy(k)

    def body(step, _):
        slot = jax.lax.rem(step, NBUF)
        pltpu.make_async_copy(
            adj_hbm.at[pl.ds(step * BM, BM), :],
            ring.at[slot],
            sems.at[slot],
        ).wait()
        out_ref[pl.ds(step * BM, BM), :] = (
            jnp.dot(ring[slot], support_ref[...], preferred_element_type=jnp.float32)
            + b_ref[...]
        )

        @pl.when(step + NBUF < NSTEPS)
        def _():
            start_copy(step + NBUF)

        return 0

    jax.lax.fori_loop(0, NSTEPS, body, 0)


@jax.jit
def kernel(infeatn, adj, W, b):
    b2 = b.reshape(1, D_OUT)
    return pl.pallas_call(
        _gcn_kernel,
        in_specs=[
            pl.BlockSpec(memory_space=pltpu.VMEM),
            pl.BlockSpec(memory_space=pl.ANY),
            pl.BlockSpec(memory_space=pltpu.VMEM),
            pl.BlockSpec(memory_space=pltpu.VMEM),
        ],
        out_specs=pl.BlockSpec(memory_space=pltpu.VMEM),
        out_shape=jax.ShapeDtypeStruct((N, D_OUT), jnp.float32),
        scratch_shapes=[
            pltpu.VMEM((NBUF, BM, N), jnp.float32),
            pltpu.VMEM((N, D_OUT), jnp.float32),
            pltpu.SemaphoreType.DMA((NBUF,)),
        ],
    )(infeatn, adj, W, b2)


# speedup vs baseline: 1.0861x; 1.0003x over previous
import jax
import jax.numpy as jnp
from jax.experimental import pallas as pl
from jax.experimental.pallas import tpu as pltpu

N = 4096
D_IN = 64
D_OUT = 64
BM = 256
NBUF = 8
NSTEPS = N // BM


def _gcn_kernel(
    infeatn_hbm, adj_hbm, w_ref, b_ref, out_hbm,
    ring, infeatn_vmem, support_ref, out_stage,
    adj_sems, in_sem, out_sems,
):
    pltpu.make_async_copy(infeatn_hbm, infeatn_vmem, in_sem).start()

    def start_copy(step):
        slot = jax.lax.rem(step, NBUF)
        pltpu.make_async_copy(
            adj_hbm.at[pl.ds(step * BM, BM), :],
            ring.at[slot],
            adj_sems.at[slot],
        ).start()

    for k in range(NBUF):
        start_copy(k)

    pltpu.make_async_copy(infeatn_hbm, infeatn_vmem, in_sem).wait()
    support_ref[...] = jnp.dot(
        infeatn_vmem[...], w_ref[...], preferred_element_type=jnp.float32
    )

    def body(step, _):
        slot = jax.lax.rem(step, NBUF)
        pltpu.make_async_copy(
            adj_hbm.at[pl.ds(step * BM, BM), :],
            ring.at[slot],
            adj_sems.at[slot],
        ).wait()

        @pl.when(step >= NBUF)
        def _():
            pltpu.make_async_copy(
                out_stage.at[slot],
                out_hbm.at[pl.ds((step - NBUF) * BM, BM), :],
                out_sems.at[slot],
            ).wait()

        out_stage[slot] = (
            jnp.dot(ring[slot], support_ref[...], preferred_element_type=jnp.float32)
            + b_ref[...]
        )
        pltpu.make_async_copy(
            out_stage.at[slot],
            out_hbm.at[pl.ds(step * BM, BM), :],
            out_sems.at[slot],
        ).start()

        @pl.when(step + NBUF < NSTEPS)
        def _():
            start_copy(step + NBUF)

        return 0

    jax.lax.fori_loop(0, NSTEPS, body, 0)

    def drain(step, _):
        slot = jax.lax.rem(step, NBUF)
        pltpu.make_async_copy(
            out_stage.at[slot],
            out_hbm.at[pl.ds(step * BM, BM), :],
            out_sems.at[slot],
        ).wait()
        return 0

    jax.lax.fori_loop(NSTEPS - NBUF, NSTEPS, drain, 0)


@jax.jit
def kernel(infeatn, adj, W, b):
    b2 = b.reshape(1, D_OUT)
    return pl.pallas_call(
        _gcn_kernel,
        in_specs=[
            pl.BlockSpec(memory_space=pl.ANY),
            pl.BlockSpec(memory_space=pl.ANY),
            pl.BlockSpec(memory_space=pltpu.VMEM),
            pl.BlockSpec(memory_space=pltpu.VMEM),
        ],
        out_specs=pl.BlockSpec(memory_space=pl.ANY),
        out_shape=jax.ShapeDtypeStruct((N, D_OUT), jnp.float32),
        scratch_shapes=[
            pltpu.VMEM((NBUF, BM, N), jnp.float32),
            pltpu.VMEM((N, D_IN), jnp.float32),
            pltpu.VMEM((N, D_OUT), jnp.float32),
            pltpu.VMEM((NBUF, BM, D_OUT), jnp.float32),
            pltpu.SemaphoreType.DMA((NBUF,)),
            pltpu.SemaphoreType.DMA,
            pltpu.SemaphoreType.DMA((NBUF,)),
        ],
    )(infeatn, adj, W, b2)
